# baseline (speedup 1.0000x reference)
import dataclasses

import jax
import jax.numpy as jnp
from jax import lax
from jax.experimental import pallas as pl
from jax.experimental.pallas import tpu as pltpu
from jax.experimental.pallas import tpu_sc as plsc

B, K, F, N = 16, 512, 256, 1024
L = 16
NC, NS = 2, 16
HALF = K // 2
CH = 128
NCHUNK = HALF // CH


def _unpool_body(x_hbm, idx_hbm, z_hbm, out_hbm, idx_v, pos_v, npos_v,
                 data0_v, data1_v, zero_v, sem_r0, sem_r1, sem_rz, sem_w):
    wid = lax.axis_index("s") * NC + lax.axis_index("c")
    b = wid // 2
    h = wid % 2
    lo = h * HALF
    row0 = b * K + lo

    rd0 = pltpu.async_copy(x_hbm.at[pl.ds(row0, CH)], data0_v, sem_r0)
    rd1 = pltpu.async_copy(x_hbm.at[pl.ds(row0 + CH, CH)], data1_v, sem_r1)
    rdz = pltpu.async_copy(z_hbm, zero_v, sem_rz)

    pltpu.sync_copy(idx_hbm.at[b], idx_v)

    def scan_mask(c, nsel):
        mv = idx_v[pl.ds(c * L, L)]
        m = mv != 0
        incl = nsel + jnp.cumsum(m.astype(jnp.int32))
        prow = c * L + lax.iota(jnp.int32, L)
        dest = b * N + prow
        r = incl - 1
        ru = prow - incl
        sel_in = m & (r >= lo) & (r < lo + HALF)
        uns_in = (~m) & (ru >= lo) & (ru < lo + HALF)
        lr = jnp.clip(r - lo, 0, HALF - 1)
        lru = jnp.clip(ru - lo, 0, HALF - 1)
        plsc.store_scatter(pos_v, [lr // CH, lr % CH], dest, mask=sel_in)
        plsc.store_scatter(npos_v, [lru // CH, lru % CH], dest, mask=uns_in)
        return incl[L - 1]

    lax.fori_loop(0, N // L, scan_mask, jnp.int32(0))

    rdz.wait()
    wz0 = pltpu.async_copy(zero_v, out_hbm.at[npos_v.at[0]], sem_w)
    wz1 = pltpu.async_copy(zero_v, out_hbm.at[npos_v.at[1]], sem_w)
    rd0.wait()
    wd0 = pltpu.async_copy(data0_v, out_hbm.at[pos_v.at[0]], sem_w)
    rd1.wait()
    wd1 = pltpu.async_copy(data1_v, out_hbm.at[pos_v.at[1]], sem_w)
    wz0.wait()
    wz1.wait()
    wd0.wait()
    wd1.wait()


def kernel(x, idx, A):
    del A
    x_flat = x.reshape(B * K, F)
    mesh = plsc.VectorSubcoreMesh(core_axis_name="c", subcore_axis_name="s")
    cp = pltpu.CompilerParams()
    if "needs_layout_passes" in pltpu.CompilerParams.__dataclass_fields__:
        cp = dataclasses.replace(cp, needs_layout_passes=False)
    out = pl.kernel(
        _unpool_body,
        out_type=jax.ShapeDtypeStruct((B * N, F), jnp.float32),
        mesh=mesh,
        compiler_params=cp,
        scratch_types=[
            pltpu.VMEM((N,), jnp.int32),
            pltpu.VMEM((NCHUNK, CH), jnp.int32),
            pltpu.VMEM((NCHUNK, CH), jnp.int32),
            pltpu.VMEM((CH, F), jnp.float32),
            pltpu.VMEM((CH, F), jnp.float32),
            pltpu.VMEM((CH, F), jnp.float32),
            pltpu.SemaphoreType.DMA,
            pltpu.SemaphoreType.DMA,
            pltpu.SemaphoreType.DMA,
            pltpu.SemaphoreType.DMA,
        ],
    )(x_flat, idx.astype(jnp.int32), jnp.zeros((CH, F), jnp.float32))
    return out.reshape(B, N, F)


# speedup vs baseline: 1.2399x; 1.2399x over previous
import dataclasses

import jax
import jax.numpy as jnp
from jax import lax
from jax.experimental import pallas as pl
from jax.experimental.pallas import tpu as pltpu
from jax.experimental.pallas import tpu_sc as plsc

B, K, F, N = 16, 512, 256, 1024
L = 16
NC, NS = 2, 16
HALF = K // 2
CH = 128
NCHUNK = HALF // CH
CZ = 64
NZCHUNK = HALF // CZ


def _unpool_body(x_hbm, idx_hbm, out_hbm, idx_v, pos_v, npos_v,
                 data0_v, data1_v, zero_v, sem_r0, sem_r1, sem_w):
    wid = lax.axis_index("s") * NC + lax.axis_index("c")
    b = wid // 2
    h = wid % 2
    lo = h * HALF
    row0 = b * K + lo

    rd0 = pltpu.async_copy(x_hbm.at[pl.ds(row0, CH)], data0_v, sem_r0)
    rd1 = pltpu.async_copy(x_hbm.at[pl.ds(row0 + CH, CH)], data1_v, sem_r1)

    pltpu.sync_copy(idx_hbm.at[b], idx_v)

    zvec = jnp.zeros((L,), jnp.float32)

    @pl.loop(0, CZ)
    def _(r):
        for c in range(0, F, L):
            zero_v[r, pl.ds(c, L)] = zvec

    def scan_mask(c, nsel):
        mv = idx_v[pl.ds(c * L, L)]
        m = mv != 0
        incl = nsel + jnp.cumsum(m.astype(jnp.int32))
        prow = c * L + lax.iota(jnp.int32, L)
        dest = b * N + prow
        r = incl - 1
        ru = prow - incl
        sel_in = m & (r >= lo) & (r < lo + HALF)
        uns_in = (~m) & (ru >= lo) & (ru < lo + HALF)
        lr = jnp.clip(r - lo, 0, HALF - 1)
        lru = jnp.clip(ru - lo, 0, HALF - 1)
        plsc.store_scatter(pos_v, [lr // CH, lr % CH], dest, mask=sel_in)
        plsc.store_scatter(npos_v, [lru // CZ, lru % CZ], dest, mask=uns_in)
        return incl[L - 1]

    lax.fori_loop(0, N // L, scan_mask, jnp.int32(0))

    wz = [pltpu.async_copy(zero_v, out_hbm.at[npos_v.at[j]], sem_w)
          for j in range(NZCHUNK)]
    rd0.wait()
    wd0 = pltpu.async_copy(data0_v, out_hbm.at[pos_v.at[0]], sem_w)
    rd1.wait()
    wd1 = pltpu.async_copy(data1_v, out_hbm.at[pos_v.at[1]], sem_w)
    for w in wz:
        w.wait()
    wd0.wait()
    wd1.wait()


def kernel(x, idx, A):
    del A
    x_flat = x.reshape(B * K, F)
    mesh = plsc.VectorSubcoreMesh(core_axis_name="c", subcore_axis_name="s")
    cp = pltpu.CompilerParams()
    if "needs_layout_passes" in pltpu.CompilerParams.__dataclass_fields__:
        cp = dataclasses.replace(cp, needs_layout_passes=False)
    out = pl.kernel(
        _unpool_body,
        out_type=jax.ShapeDtypeStruct((B * N, F), jnp.float32),
        mesh=mesh,
        compiler_params=cp,
        scratch_types=[
            pltpu.VMEM((N,), jnp.int32),
            pltpu.VMEM((NCHUNK, CH), jnp.int32),
            pltpu.VMEM((NZCHUNK, CZ), jnp.int32),
            pltpu.VMEM((CH, F), jnp.float32),
            pltpu.VMEM((CH, F), jnp.float32),
            pltpu.VMEM((CZ, F), jnp.float32),
            pltpu.SemaphoreType.DMA,
            pltpu.SemaphoreType.DMA,
            pltpu.SemaphoreType.DMA,
        ],
    )(x_flat, idx.astype(jnp.int32))
    return out.reshape(B, N, F)


# speedup vs baseline: 1.2459x; 1.0049x over previous
import dataclasses

import jax
import jax.numpy as jnp
from jax import lax
from jax.experimental import pallas as pl
from jax.experimental.pallas import tpu as pltpu
from jax.experimental.pallas import tpu_sc as plsc

B, K, F, N = 16, 512, 256, 1024
L = 16
NC, NS = 2, 16
HALF = K // 2
CH = 128
NCHUNK = HALF // CH
CZ = 64
NZCHUNK = HALF // CZ


def _unpool_body(x_hbm, idx_hbm, out_hbm, idx_v, pos_v, npos_v,
                 data0_v, data1_v, zero_v, sem_r0, sem_r1, sem_w):
    wid = lax.axis_index("s") * NC + lax.axis_index("c")
    b = wid // 2
    h = wid % 2
    lo = h * HALF
    row0 = b * K + lo

    rd0 = pltpu.async_copy(x_hbm.at[pl.ds(row0, CH)], data0_v, sem_r0)
    rd1 = pltpu.async_copy(x_hbm.at[pl.ds(row0 + CH, CH)], data1_v, sem_r1)

    pltpu.sync_copy(idx_hbm.at[b], idx_v)

    zvec = jnp.zeros((L,), jnp.float32)

    @pl.loop(0, CZ)
    def _(r):
        @pl.loop(0, F, step=L)
        def _(c):
            zero_v[r, pl.ds(c, L)] = zvec

    def scan_mask(c, nsel):
        mv = idx_v[pl.ds(c * L, L)]
        m = mv != 0
        incl = nsel + jnp.cumsum(m.astype(jnp.int32))
        prow = c * L + lax.iota(jnp.int32, L)
        dest = b * N + prow
        r = incl - 1
        ru = prow - incl
        sel_in = m & (r >= lo) & (r < lo + HALF)
        uns_in = (~m) & (ru >= lo) & (ru < lo + HALF)
        lr = jnp.clip(r - lo, 0, HALF - 1)
        lru = jnp.clip(ru - lo, 0, HALF - 1)
        plsc.store_scatter(pos_v, [lr // CH, lr % CH], dest, mask=sel_in)
        plsc.store_scatter(npos_v, [lru // CZ, lru % CZ], dest, mask=uns_in)
        return incl[L - 1]

    lax.fori_loop(0, N // L, scan_mask, jnp.int32(0))

    wz = [pltpu.async_copy(zero_v, out_hbm.at[npos_v.at[j]], sem_w)
          for j in range(NZCHUNK)]
    rd0.wait()
    wd0 = pltpu.async_copy(data0_v, out_hbm.at[pos_v.at[0]], sem_w)
    rd1.wait()
    wd1 = pltpu.async_copy(data1_v, out_hbm.at[pos_v.at[1]], sem_w)
    for w in wz:
        w.wait()
    wd0.wait()
    wd1.wait()


def kernel(x, idx, A):
    del A
    x_flat = x.reshape(B * K, F)
    mesh = plsc.VectorSubcoreMesh(core_axis_name="c", subcore_axis_name="s")
    cp = pltpu.CompilerParams()
    if "needs_layout_passes" in pltpu.CompilerParams.__dataclass_fields__:
        cp = dataclasses.replace(cp, needs_layout_passes=False)
    out = pl.kernel(
        _unpool_body,
        out_type=jax.ShapeDtypeStruct((B * N, F), jnp.float32),
        mesh=mesh,
        compiler_params=cp,
        scratch_types=[
            pltpu.VMEM((N,), jnp.int32),
            pltpu.VMEM((NCHUNK, CH), jnp.int32),
            pltpu.VMEM((NZCHUNK, CZ), jnp.int32),
            pltpu.VMEM((CH, F), jnp.float32),
            pltpu.VMEM((CH, F), jnp.float32),
            pltpu.VMEM((CZ, F), jnp.float32),
            pltpu.SemaphoreType.DMA,
            pltpu.SemaphoreType.DMA,
            pltpu.SemaphoreType.DMA,
        ],
    )(x_flat, idx.astype(jnp.int32))
    return out.reshape(B, N, F)
